# baseline (speedup 1.0000x reference)
import functools

import jax
import jax.numpy as jnp
from jax import lax
from jax.experimental import pallas as pl
from jax.experimental.pallas import tpu as pltpu
from jax.experimental.pallas import tpu_sc as plsc

_N_ROWS = 4096
_N_COLS = 20000
_N_SEL = 2000
_NC = 2
_NS = 16
_L = 16
_NW = _NC * _NS
_ROWS_PER_W = _N_ROWS // _NW
_MASK_VREGS = _N_COLS // _L
_SEL_VREGS = _N_SEL // _L


def _body(x_hbm, mask_hbm, out_hbm, mask_v, idx_v, row_v, orow_v):
    wid = lax.axis_index("s") * _NC + lax.axis_index("c")

    pltpu.sync_copy(mask_hbm, mask_v)

    def zero_body(j, carry):
        idx_v[pl.ds(j * _L, _L)] = jnp.zeros((_L,), jnp.int32)
        return carry

    lax.fori_loop(0, _SEL_VREGS, zero_body, jnp.int32(0), unroll=False)

    def ph1(j, off):
        mv = mask_v[pl.ds(j * _L, _L)]
        m = mv != 0.0
        mi = m.astype(jnp.int32)
        c = plsc.cumsum(mi)
        pos = off + c - 1
        vals = lax.iota(jnp.int32, _L) + j * _L
        safe = m & (pos < _N_SEL)
        plsc.store_scatter(idx_v, [pos], vals, mask=safe)
        return off + jnp.sum(mi)

    lax.fori_loop(0, _MASK_VREGS, ph1, jnp.int32(0), unroll=False)

    def ph2(r, carry):
        row = wid * _ROWS_PER_W + r
        pltpu.sync_copy(x_hbm.at[row], row_v)

        def inner(j, c2):
            iv = idx_v[pl.ds(j * _L, _L)]
            orow_v[pl.ds(j * _L, _L)] = plsc.load_gather(row_v, [iv])
            return c2

        lax.fori_loop(0, _SEL_VREGS, inner, jnp.int32(0), unroll=False)
        pltpu.sync_copy(orow_v, out_hbm.at[row])
        return carry

    lax.fori_loop(0, _ROWS_PER_W, ph2, jnp.int32(0), unroll=False)


@functools.partial(
    pl.kernel,
    out_type=jax.ShapeDtypeStruct((_N_ROWS, _N_SEL), jnp.float32),
    mesh=plsc.VectorSubcoreMesh(core_axis_name="c", subcore_axis_name="s"),
    compiler_params=pltpu.CompilerParams(needs_layout_passes=False),
    scratch_types=[
        pltpu.VMEM((_N_COLS,), jnp.float32),
        pltpu.VMEM((_N_SEL,), jnp.int32),
        pltpu.VMEM((_N_COLS,), jnp.float32),
        pltpu.VMEM((_N_SEL,), jnp.float32),
    ],
)
def _gather_columns(x_hbm, mask_hbm, out_hbm, mask_v, idx_v, row_v, orow_v):
    _body(x_hbm, mask_hbm, out_hbm, mask_v, idx_v, row_v, orow_v)


def kernel(x, selection_mask):
    return _gather_columns(x, selection_mask)


# speedup vs baseline: 10.1985x; 10.1985x over previous
import functools

import jax
import jax.numpy as jnp
from jax import lax
from jax.experimental import pallas as pl
from jax.experimental.pallas import tpu as pltpu
from jax.experimental.pallas import tpu_sc as plsc

_N_ROWS = 4096
_N_COLS = 20000
_N_SEL = 2000
_NC = 2
_NS = 16
_L = 16
_NW = _NC * _NS
_MASK_VREGS = _N_COLS // _L
_IDX_PAD = 2048
_K = 8
_NCHUNK = 8


def _body(xt_hbm, mask_hbm, out_hbm, mask_v, idx_v, buf0, buf1, sem0, sem1):
    wid = lax.axis_index("s") * _NC + lax.axis_index("c")

    pltpu.sync_copy(mask_hbm, mask_v)

    def zero_body(j, carry):
        idx_v[pl.ds(j * _L, _L)] = jnp.zeros((_L,), jnp.int32)
        return carry

    lax.fori_loop(0, _IDX_PAD // _L, zero_body, jnp.int32(0), unroll=False)

    def ph1(j, off):
        mv = mask_v[pl.ds(j * _L, _L)]
        m = mv != 0.0
        mi = m.astype(jnp.int32)
        c = plsc.cumsum(mi)
        pos = off + c - 1
        vals = lax.iota(jnp.int32, _L) + j * _L
        safe = m & (pos < _N_SEL)
        plsc.store_scatter(idx_v, [pos], vals, mask=safe)
        return off + jnp.sum(mi)

    lax.fori_loop(0, _MASK_VREGS, ph1, jnp.int32(0), unroll=False)

    base = wid * (_IDX_PAD // _NW)
    starts = [jnp.minimum(base + _K * c, _N_SEL - _K) for c in range(_NCHUNK)]
    bufs = [buf0, buf1]
    sems = [sem0, sem1]

    def gather_start(c):
        return pltpu.async_copy(
            xt_hbm.at[idx_v.at[pl.ds(starts[c], _K)]], bufs[c % 2], sems[c % 2]
        )

    cp = gather_start(0)
    for c in range(_NCHUNK):
        cp.wait()
        if c + 1 < _NCHUNK:
            nxt = gather_start(c + 1)
        pltpu.sync_copy(bufs[c % 2], out_hbm.at[pl.ds(starts[c], _K), :])
        if c + 1 < _NCHUNK:
            cp = nxt


@functools.partial(
    pl.kernel,
    out_type=jax.ShapeDtypeStruct((_N_SEL, _N_ROWS), jnp.float32),
    mesh=plsc.VectorSubcoreMesh(core_axis_name="c", subcore_axis_name="s"),
    compiler_params=pltpu.CompilerParams(needs_layout_passes=False),
    scratch_types=[
        pltpu.VMEM((_N_COLS,), jnp.float32),
        pltpu.VMEM((_IDX_PAD,), jnp.int32),
        pltpu.VMEM((_K, _N_ROWS), jnp.float32),
        pltpu.VMEM((_K, _N_ROWS), jnp.float32),
        pltpu.SemaphoreType.DMA,
        pltpu.SemaphoreType.DMA,
    ],
)
def _gather_rows_t(xt_hbm, mask_hbm, out_hbm, mask_v, idx_v, buf0, buf1, sem0, sem1):
    _body(xt_hbm, mask_hbm, out_hbm, mask_v, idx_v, buf0, buf1, sem0, sem1)


def kernel(x, selection_mask):
    xt = jnp.transpose(x)
    out_t = _gather_rows_t(xt, selection_mask)
    return jnp.transpose(out_t)
